# Initial kernel scaffold; baseline (speedup 1.0000x reference)
import jax
import jax.numpy as jnp
from jax.experimental import pallas as pl


def kernel(x, edge_index, batch, W1, b1, W2, b2, W4, b4, Wd1, bd1, Wd2, bd2, Wo, bo):
    raise NotImplementedError("write your pallas kernel here")


# speedup vs baseline: 10.9560x; 10.9560x over previous
import functools

import jax
import jax.numpy as jnp
from jax import lax
from jax.experimental import pallas as pl
from jax.experimental.pallas import tpu as pltpu
from jax.experimental.pallas import tpu_sc as plsc

N_NODES = 10000
N_PAD = 10240
N_EDGES = 320000
D = 128
N_GRAPHS = 64

NC = 2
NS = 16
NW = NC * NS
E_PER_TILE = N_EDGES // NW
CHUNK = 80
STEPS = E_PER_TILE // CHUNK
ROWS_PER_TILE = N_PAD // NS

R_BLK = 2048
GRID = N_PAD // R_BLK


def _sc_mesh():
    return plsc.VectorSubcoreMesh(core_axis_name="c", subcore_axis_name="s",
                                  num_cores=NC, num_subcores=NS)


@functools.cache
def _make_deg_sc():
    return functools.partial(
        pl.kernel,
        out_type=jax.ShapeDtypeStruct((NC, N_PAD, 16), jnp.float32),
        mesh=_sc_mesh(),
        scratch_types=[
            pltpu.VMEM((1, CHUNK), jnp.int32),
            pltpu.VMEM((CHUNK, 16), jnp.float32),
            pltpu.VMEM_SHARED((N_PAD, 16), jnp.float32),
            pltpu.SemaphoreType.DMA,
        ],
    )(_deg_sc_body)


def _deg_sc_body(col_hbm, zeros_hbm, out_hbm, colv, ones_v, acc, sem):
    c = lax.axis_index("c")
    s = lax.axis_index("s")
    wid = c * NS + s
    r0 = s * ROWS_PER_TILE

    def fill_ones(i, carry):
        ones_v[i, :] = jnp.ones((16,), jnp.float32)
        return carry

    lax.fori_loop(0, CHUNK, fill_ones, 0)
    pltpu.sync_copy(zeros_hbm, acc.at[pl.ds(r0, ROWS_PER_TILE)])
    plsc.subcore_barrier()

    def body(j, carry):
        base = wid * E_PER_TILE + j * CHUNK
        pltpu.sync_copy(col_hbm.at[pl.ds(base, CHUNK)], colv.at[0])
        pltpu.sync_copy(ones_v, acc.at[colv.at[0]], add=True)
        return carry

    lax.fori_loop(0, STEPS, body, 0)
    plsc.subcore_barrier()
    pltpu.sync_copy(acc.at[pl.ds(r0, ROWS_PER_TILE)],
                    out_hbm.at[c, pl.ds(r0, ROWS_PER_TILE)])


@functools.cache
def _make_prop_sc():
    return functools.partial(
        pl.kernel,
        out_type=jax.ShapeDtypeStruct((NC, N_PAD, D), jnp.float32),
        mesh=_sc_mesh(),
        scratch_types=[
            pltpu.VMEM((CHUNK,), jnp.int32),
            pltpu.VMEM((1, CHUNK), jnp.int32),
            pltpu.VMEM((CHUNK, D), jnp.float32),
            pltpu.VMEM_SHARED((N_PAD, D), jnp.float32),
            pltpu.SemaphoreType.DMA,
        ],
    )(_prop_sc_body)


def _prop_sc_body(p_hbm, row_hbm, col_hbm, zeros_hbm, out_hbm,
                  rowv, colv, gbuf, acc, gsem):
    c = lax.axis_index("c")
    s = lax.axis_index("s")
    wid = c * NS + s
    r0 = s * ROWS_PER_TILE

    pltpu.sync_copy(zeros_hbm, acc.at[pl.ds(r0, ROWS_PER_TILE)])
    plsc.subcore_barrier()

    def body(j, carry):
        base = wid * E_PER_TILE + j * CHUNK
        pltpu.sync_copy(row_hbm.at[pl.ds(base, CHUNK)], rowv)
        pltpu.sync_copy(col_hbm.at[pl.ds(base, CHUNK)], colv.at[0])
        pltpu.async_copy(p_hbm.at[rowv], gbuf, gsem).wait()
        pltpu.sync_copy(gbuf, acc.at[colv.at[0]], add=True)
        return carry

    lax.fori_loop(0, STEPS, body, 0)
    plsc.subcore_barrier()
    pltpu.sync_copy(acc.at[pl.ds(r0, ROWS_PER_TILE)],
                    out_hbm.at[c, pl.ds(r0, ROWS_PER_TILE)])


def _mm1_body(x_ref, w_ref, d0_ref, d1_ref, p_ref, dis_ref):
    deg = d0_ref[:, 0:1] + d1_ref[:, 0:1] + 1.0
    dis = lax.rsqrt(deg)
    h = jnp.dot(x_ref[...], w_ref[...], preferred_element_type=jnp.float32)
    p_ref[...] = h * dis
    dis_ref[...] = dis


def _mm1(x, W1, deg0, deg1):
    return pl.pallas_call(
        _mm1_body,
        grid=(GRID,),
        in_specs=[
            pl.BlockSpec((R_BLK, D), lambda i: (i, 0)),
            pl.BlockSpec((D, D), lambda i: (0, 0)),
            pl.BlockSpec((R_BLK, 16), lambda i: (i, 0)),
            pl.BlockSpec((R_BLK, 16), lambda i: (i, 0)),
        ],
        out_specs=[
            pl.BlockSpec((R_BLK, D), lambda i: (i, 0)),
            pl.BlockSpec((R_BLK, 1), lambda i: (i, 0)),
        ],
        out_shape=[
            jax.ShapeDtypeStruct((N_PAD, D), jnp.float32),
            jax.ShapeDtypeStruct((N_PAD, 1), jnp.float32),
        ],
    )(x, W1, deg0, deg1)


def _combine_body(s0_ref, s1_ref, p_ref, dis_ref, b_ref, w_ref, pn_ref):
    dis = dis_ref[...]
    a = dis * (s0_ref[...] + s1_ref[...] + p_ref[...]) + b_ref[...]
    a = jnp.maximum(a, 0.0)
    pn_ref[...] = dis * jnp.dot(a, w_ref[...],
                                preferred_element_type=jnp.float32)


def _combine_mm(s_parts, p, dis, b, Wn):
    return pl.pallas_call(
        _combine_body,
        grid=(GRID,),
        in_specs=[
            pl.BlockSpec((R_BLK, D), lambda i: (i, 0)),
            pl.BlockSpec((R_BLK, D), lambda i: (i, 0)),
            pl.BlockSpec((R_BLK, D), lambda i: (i, 0)),
            pl.BlockSpec((R_BLK, 1), lambda i: (i, 0)),
            pl.BlockSpec((1, D), lambda i: (0, 0)),
            pl.BlockSpec((D, D), lambda i: (0, 0)),
        ],
        out_specs=pl.BlockSpec((R_BLK, D), lambda i: (i, 0)),
        out_shape=jax.ShapeDtypeStruct((N_PAD, D), jnp.float32),
    )(s_parts[0], s_parts[1], p, dis, b.reshape(1, D), Wn)


def _final_body(s0_ref, s1_ref, p_ref, dis_ref, b4_ref, batch_ref,
                wd1_ref, bd1_ref, wd2_ref, bd2_ref, wo_ref, bo_ref,
                out_ref, sums, counts):
    i = pl.program_id(0)

    @pl.when(i == 0)
    def _():
        sums[...] = jnp.zeros_like(sums)
        counts[...] = jnp.zeros_like(counts)

    h = dis_ref[...] * (s0_ref[...] + s1_ref[...] + p_ref[...]) + b4_ref[...]
    bb = batch_ref[0, 0, :]
    onehot = (bb[None, :] ==
              lax.broadcasted_iota(jnp.int32, (N_GRAPHS, R_BLK), 0)
              ).astype(jnp.float32)
    sums[...] += jnp.dot(onehot, h, preferred_element_type=jnp.float32)
    counts[...] += jnp.sum(onehot, axis=1, keepdims=True)

    @pl.when(i == pl.num_programs(0) - 1)
    def _():
        g = sums[...] / jnp.maximum(counts[...], 1.0)
        g = jnp.maximum(
            jnp.dot(g, wd1_ref[...], preferred_element_type=jnp.float32)
            + bd1_ref[...], 0.0)
        g = jnp.maximum(
            jnp.dot(g, wd2_ref[...], preferred_element_type=jnp.float32)
            + bd2_ref[...], 0.0)
        out_ref[...] = (jnp.dot(g, wo_ref[...],
                                preferred_element_type=jnp.float32)
                        + bo_ref[...])


def _final(s_parts, p, dis, b4, batch_r, Wd1, bd1, Wd2, bd2, Wo, bo):
    return pl.pallas_call(
        _final_body,
        grid=(GRID,),
        in_specs=[
            pl.BlockSpec((R_BLK, D), lambda i: (i, 0)),
            pl.BlockSpec((R_BLK, D), lambda i: (i, 0)),
            pl.BlockSpec((R_BLK, D), lambda i: (i, 0)),
            pl.BlockSpec((R_BLK, 1), lambda i: (i, 0)),
            pl.BlockSpec((1, D), lambda i: (0, 0)),
            pl.BlockSpec((1, 1, R_BLK), lambda i: (i, 0, 0)),
            pl.BlockSpec((D, D), lambda i: (0, 0)),
            pl.BlockSpec((1, D), lambda i: (0, 0)),
            pl.BlockSpec((D, D), lambda i: (0, 0)),
            pl.BlockSpec((1, D), lambda i: (0, 0)),
            pl.BlockSpec((D, 1), lambda i: (0, 0)),
            pl.BlockSpec((1, 1), lambda i: (0, 0)),
        ],
        out_specs=pl.BlockSpec((N_GRAPHS, 1), lambda i: (0, 0)),
        out_shape=jax.ShapeDtypeStruct((N_GRAPHS, 1), jnp.float32),
        scratch_shapes=[
            pltpu.VMEM((N_GRAPHS, D), jnp.float32),
            pltpu.VMEM((N_GRAPHS, 1), jnp.float32),
        ],
    )(s_parts[0], s_parts[1], p, dis, b4.reshape(1, D), batch_r,
      Wd1, bd1.reshape(1, D), Wd2, bd2.reshape(1, D), Wo, bo.reshape(1, 1))


def kernel(x, edge_index, batch, W1, b1, W2, b2, W4, b4,
           Wd1, bd1, Wd2, bd2, Wo, bo):
    ei = edge_index.astype(jnp.int32)
    row = ei[0]
    col = ei[1]
    x = jnp.pad(x, ((0, N_PAD - N_NODES), (0, 0)))
    batch_r = jnp.pad(batch.astype(jnp.int32), (0, N_PAD - N_NODES),
                      constant_values=N_GRAPHS + 1).reshape(GRID, 1, R_BLK)

    zeros_deg = jnp.zeros((ROWS_PER_TILE, 16), jnp.float32)
    zeros_d = jnp.zeros((ROWS_PER_TILE, D), jnp.float32)

    deg_parts = _make_deg_sc()(col, zeros_deg)
    p1, dis = _mm1(x, W1, deg_parts[0], deg_parts[1])

    prop = _make_prop_sc()
    s1 = prop(p1, row, col, zeros_d)
    p2 = _combine_mm(s1, p1, dis, b1, W2)
    s2 = prop(p2, row, col, zeros_d)
    p3 = _combine_mm(s2, p2, dis, b2, W4)
    s3 = prop(p3, row, col, zeros_d)

    return _final(s3, p3, dis, b4, batch_r, Wd1, bd1, Wd2, bd2, Wo, bo)
